# baseline (speedup 1.0000x reference)
import functools

import jax
import jax.numpy as jnp
from jax import lax
from jax.experimental import pallas as pl
from jax.experimental.pallas import tpu as pltpu
from jax.experimental.pallas import tpu_sc as plsc

VOCAB = 1000000
DIM = 64
BATCH = 16384
CTX_LEN = 20
NUM_NEG = 5

NC = 2
NS = 16
NW = NC * NS
LANES = 16
DV = DIM // LANES

B_PER_W = BATCH // NW
CHUNK_A = 32
N_CHUNK_A = B_PER_W // CHUNK_A
CHUNK_B = 128
N_CHUNK_B = B_PER_W // CHUNK_B
IDX_ROWS = 5
ROWS_PER_CHUNK = 640

_GDN = lax.GatherDimensionNumbers(
    offset_dims=(), collapsed_slice_dims=(0,), start_index_map=(0,))


def _lane_shuffle(x, idx):
  return lax.gather(x, idx[:, None], _GDN, (1,),
                    mode=lax.GatherScatterMode.PROMISE_IN_BOUNDS)


def _sc_body(ctx_idx_hbm, neg_idx_hbm, embed_hbm, context_hbm, out_hbm,
             idx_v, rows_a, rows_b, mean_v, out_v, sem_a, sem_b):
  wid = lax.axis_index("s") * NC + lax.axis_index("c")
  lane = lax.iota(jnp.int32, LANES)
  shuf_idx = [lane ^ s for s in (8, 4, 2, 1)]
  k_masks = [lane == k for k in range(NUM_NEG)]

  rows_bufs = (rows_a, rows_b)
  sems = (sem_a, sem_b)

  def load_idx(src_hbm, off0, buf):
    for j in range(IDX_ROWS):
      pltpu.sync_copy(src_hbm.at[pl.ds(off0 + j * 128, 128)],
                      idx_v.at[buf, j])

  def fire_gathers(table_hbm, buf):
    handles = []
    for j in range(IDX_ROWS):
      handles.append(pltpu.async_copy(
          table_hbm.at[idx_v.at[buf, j]],
          rows_bufs[buf].at[pl.ds(j * 128, 128), :],
          sems[buf]))
    return handles

  def drain(handles):
    for h in handles:
      h.wait()

  ctx_off0 = wid * (B_PER_W * CTX_LEN)

  load_idx(ctx_idx_hbm, ctx_off0, 0)
  pending = {0: fire_gathers(embed_hbm, 0)}
  for g in range(N_CHUNK_A):
    cur = g % 2
    if g + 1 < N_CHUNK_A:
      nxt = (g + 1) % 2
      load_idx(ctx_idx_hbm, ctx_off0 + (g + 1) * IDX_ROWS * 128, nxt)
      pending[nxt] = fire_gathers(embed_hbm, nxt)
    drain(pending.pop(cur))
    rows = rows_bufs[cur]

    def mean_body(i, _, g=g, rows=rows):
      accs = [rows[i * CTX_LEN, pl.ds(q * LANES, LANES)] for q in range(DV)]
      for j in range(1, CTX_LEN):
        for q in range(DV):
          accs[q] = accs[q] + rows[i * CTX_LEN + j, pl.ds(q * LANES, LANES)]
      b = g * CHUNK_A + i
      for q in range(DV):
        mean_v[b, pl.ds(q * LANES, LANES)] = accs[q] * (1.0 / CTX_LEN)
      return _

    lax.fori_loop(0, CHUNK_A, mean_body, 0, unroll=False)

  neg_off0 = wid * (B_PER_W * NUM_NEG)
  out_base = wid * (B_PER_W * NUM_NEG)

  load_idx(neg_idx_hbm, neg_off0, 0)
  pending = {0: fire_gathers(context_hbm, 0)}
  for h in range(N_CHUNK_B):
    cur = h % 2
    if h + 1 < N_CHUNK_B:
      nxt = (h + 1) % 2
      load_idx(neg_idx_hbm, neg_off0 + (h + 1) * IDX_ROWS * 128, nxt)
      pending[nxt] = fire_gathers(context_hbm, nxt)
    drain(pending.pop(cur))
    rows = rows_bufs[cur]

    def dot_body(i, _, h=h, rows=rows):
      b = h * CHUNK_B + i
      m = [mean_v[b, pl.ds(q * LANES, LANES)] for q in range(DV)]
      acc = jnp.zeros((LANES,), jnp.float32)
      for k in range(NUM_NEG):
        r = i * NUM_NEG + k
        p = rows[r, pl.ds(0, LANES)] * m[0]
        for q in range(1, DV):
          p = p + rows[r, pl.ds(q * LANES, LANES)] * m[q]
        for sidx in shuf_idx:
          p = p + _lane_shuffle(p, sidx)
        acc = jnp.where(k_masks[k], p, acc)
      out_v[pl.ds(i * NUM_NEG, LANES)] = acc
      return _

    lax.fori_loop(0, CHUNK_B, dot_body, 0, unroll=False)
    pltpu.sync_copy(
        out_v.at[pl.ds(0, CHUNK_B * NUM_NEG)],
        out_hbm.at[pl.ds(out_base + h * CHUNK_B * NUM_NEG,
                         CHUNK_B * NUM_NEG)])


@jax.jit
def _cbow_scores(ctx_idx1d, neg_idx1d, W_embed, W_context):
  mesh = plsc.VectorSubcoreMesh(
      core_axis_name="c", subcore_axis_name="s",
      num_cores=NC, num_subcores=NS)
  fn = pl.kernel(
      _sc_body,
      out_type=jax.ShapeDtypeStruct((BATCH * NUM_NEG,), jnp.float32),
      mesh=mesh,
      compiler_params=pltpu.CompilerParams(use_tc_tiling_on_sc=False),
      scratch_types=[
          pltpu.VMEM((2, IDX_ROWS, 128), jnp.int32),
          pltpu.VMEM((ROWS_PER_CHUNK, DIM), jnp.float32),
          pltpu.VMEM((ROWS_PER_CHUNK, DIM), jnp.float32),
          pltpu.VMEM((B_PER_W, DIM), jnp.float32),
          pltpu.VMEM((CHUNK_B * NUM_NEG + LANES,), jnp.float32),
          pltpu.SemaphoreType.DMA,
          pltpu.SemaphoreType.DMA,
      ],
  )
  return fn(ctx_idx1d, neg_idx1d, W_embed, W_context)


def kernel(context_words, negative_words, W_embed, W_context):
  ctx_idx1d = context_words.reshape(BATCH * CTX_LEN)
  neg_idx1d = negative_words.reshape(BATCH * NUM_NEG)
  flat = _cbow_scores(ctx_idx1d, neg_idx1d, W_embed, W_context)
  return flat.reshape(BATCH, NUM_NEG)


# speedup vs baseline: 1.0304x; 1.0304x over previous
import functools

import jax
import jax.numpy as jnp
from jax import lax
from jax.experimental import pallas as pl
from jax.experimental.pallas import tpu as pltpu
from jax.experimental.pallas import tpu_sc as plsc

VOCAB = 1000000
DIM = 64
BATCH = 16384
CTX_LEN = 20
NUM_NEG = 5

NC = 2
NS = 16
NW = NC * NS
LANES = 16
DV = DIM // LANES

B_PER_W = BATCH // NW
CHUNK_A = 32
N_CHUNK_A = B_PER_W // CHUNK_A
CHUNK_B = 128
N_CHUNK_B = B_PER_W // CHUNK_B
IDX_ROWS = 5
ROWS_PER_CHUNK = 640

_GDN = lax.GatherDimensionNumbers(
    offset_dims=(), collapsed_slice_dims=(0,), start_index_map=(0,))


def _lane_shuffle(x, idx):
  return lax.gather(x, idx[:, None], _GDN, (1,),
                    mode=lax.GatherScatterMode.PROMISE_IN_BOUNDS)


def _sc_body(ctx_idx_hbm, neg_idx_hbm, embed_hbm, context_hbm, out_hbm,
             ctx_idx_v, neg_idx_v, rows_a, rows_b, mean_v, out_v,
             sem_a, sem_b):
  wid = lax.axis_index("s") * NC + lax.axis_index("c")
  lane = lax.iota(jnp.int32, LANES)
  shuf_idx = [lane ^ s for s in (8, 4, 2, 1)]
  k_masks = [lane == k for k in range(NUM_NEG)]

  rows_bufs = (rows_a, rows_b)
  sems = (sem_a, sem_b)

  pltpu.sync_copy(ctx_idx_hbm.at[pl.ds(wid * (B_PER_W * CTX_LEN),
                                       B_PER_W * CTX_LEN)], ctx_idx_v)
  pltpu.sync_copy(neg_idx_hbm.at[pl.ds(wid * (B_PER_W * NUM_NEG),
                                       B_PER_W * NUM_NEG)], neg_idx_v)

  def fire_gather(table_hbm, idx_ref, chunk, buf):
    return pltpu.async_copy(
        table_hbm.at[idx_ref.at[pl.ds(chunk * ROWS_PER_CHUNK,
                                      ROWS_PER_CHUNK)]],
        rows_bufs[buf], sems[buf])

  pending = {0: fire_gather(embed_hbm, ctx_idx_v, 0, 0)}
  for g in range(N_CHUNK_A):
    cur = g % 2
    if g + 1 < N_CHUNK_A:
      nxt = (g + 1) % 2
      pending[nxt] = fire_gather(embed_hbm, ctx_idx_v, g + 1, nxt)
    pending.pop(cur).wait()
    rows = rows_bufs[cur]

    def mean_body(i, _, g=g, rows=rows):
      accs = [rows[i * CTX_LEN, pl.ds(q * LANES, LANES)] for q in range(DV)]
      for j in range(1, CTX_LEN):
        for q in range(DV):
          accs[q] = accs[q] + rows[i * CTX_LEN + j, pl.ds(q * LANES, LANES)]
      b = g * CHUNK_A + i
      for q in range(DV):
        mean_v[b, pl.ds(q * LANES, LANES)] = accs[q] * (1.0 / CTX_LEN)
      return _

    lax.fori_loop(0, CHUNK_A, mean_body, 0, unroll=False)

  out_base = wid * (B_PER_W * NUM_NEG)

  pending = {0: fire_gather(context_hbm, neg_idx_v, 0, 0)}
  for h in range(N_CHUNK_B):
    cur = h % 2
    if h + 1 < N_CHUNK_B:
      nxt = (h + 1) % 2
      pending[nxt] = fire_gather(context_hbm, neg_idx_v, h + 1, nxt)
    pending.pop(cur).wait()
    rows = rows_bufs[cur]

    def dot_body(i, _, h=h, rows=rows):
      b = h * CHUNK_B + i
      m = [mean_v[b, pl.ds(q * LANES, LANES)] for q in range(DV)]
      acc = jnp.zeros((LANES,), jnp.float32)
      for k in range(NUM_NEG):
        r = i * NUM_NEG + k
        p = rows[r, pl.ds(0, LANES)] * m[0]
        for q in range(1, DV):
          p = p + rows[r, pl.ds(q * LANES, LANES)] * m[q]
        for sidx in shuf_idx:
          p = p + _lane_shuffle(p, sidx)
        acc = jnp.where(k_masks[k], p, acc)
      out_v[pl.ds(i * NUM_NEG, LANES)] = acc
      return _

    lax.fori_loop(0, CHUNK_B, dot_body, 0, unroll=False)
    pltpu.sync_copy(
        out_v.at[pl.ds(0, CHUNK_B * NUM_NEG)],
        out_hbm.at[pl.ds(out_base + h * CHUNK_B * NUM_NEG,
                         CHUNK_B * NUM_NEG)])


@jax.jit
def _cbow_scores(ctx_idx1d, neg_idx1d, W_embed, W_context):
  mesh = plsc.VectorSubcoreMesh(
      core_axis_name="c", subcore_axis_name="s",
      num_cores=NC, num_subcores=NS)
  fn = pl.kernel(
      _sc_body,
      out_type=jax.ShapeDtypeStruct((BATCH * NUM_NEG,), jnp.float32),
      mesh=mesh,
      compiler_params=pltpu.CompilerParams(use_tc_tiling_on_sc=False),
      scratch_types=[
          pltpu.VMEM((B_PER_W * CTX_LEN,), jnp.int32),
          pltpu.VMEM((B_PER_W * NUM_NEG,), jnp.int32),
          pltpu.VMEM((ROWS_PER_CHUNK, DIM), jnp.float32),
          pltpu.VMEM((ROWS_PER_CHUNK, DIM), jnp.float32),
          pltpu.VMEM((B_PER_W, DIM), jnp.float32),
          pltpu.VMEM((CHUNK_B * NUM_NEG + LANES,), jnp.float32),
          pltpu.SemaphoreType.DMA,
          pltpu.SemaphoreType.DMA,
      ],
  )
  return fn(ctx_idx1d, neg_idx1d, W_embed, W_context)


def kernel(context_words, negative_words, W_embed, W_context):
  ctx_idx1d = context_words.reshape(BATCH * CTX_LEN)
  neg_idx1d = negative_words.reshape(BATCH * NUM_NEG)
  flat = _cbow_scores(ctx_idx1d, neg_idx1d, W_embed, W_context)
  return flat.reshape(BATCH, NUM_NEG)
